# baseline (speedup 1.0000x reference)
import functools

import jax
import jax.numpy as jnp
from jax import lax
from jax.experimental import pallas as pl
from jax.experimental.pallas import tpu as pltpu
from jax.experimental.pallas import tpu_sc as plsc

VOCAB = 1000000
BATCH = 4096
SEQ = 200
DIM = 64

NUM_CORES = 2
NUM_SUBCORES = 16
NW = NUM_CORES * NUM_SUBCORES
BB = BATCH // NW
IDX_PER_W = BB * SEQ

_mesh = plsc.VectorSubcoreMesh(
    core_axis_name="c", subcore_axis_name="s",
    num_cores=NUM_CORES, num_subcores=NUM_SUBCORES,
)


@functools.partial(
    pl.kernel,
    mesh=_mesh,
    out_type=jax.ShapeDtypeStruct((SEQ, 8, NW, 8, BB), jnp.float32),
    scratch_types=[
        pltpu.VMEM((IDX_PER_W,), jnp.int32),
        pltpu.VMEM((SEQ, BB), jnp.int32),
        pltpu.VMEM((2, BB, DIM), jnp.float32),
        pltpu.VMEM((2, 1, 8, 1, 8, BB), jnp.float32),
        pltpu.SemaphoreType.DMA,
        pltpu.SemaphoreType.DMA,
        pltpu.SemaphoreType.DMA,
        pltpu.SemaphoreType.DMA,
        pltpu.SemaphoreType.DMA,
    ],
    compiler_params=pltpu.CompilerParams(
        use_tc_tiling_on_sc=False, needs_layout_passes=False),
)
def _gather_kernel(idx_hbm, table_hbm, out_hbm, idxw_v, idxt_v, wide_v,
                   tile_v, idx_sem, gat_sem0, gat_sem1, out_sem0, out_sem1):
    wid = lax.axis_index("s") * NUM_CORES + lax.axis_index("c")
    gat_sems = [gat_sem0, gat_sem1]
    out_sems = [out_sem0, out_sem1]
    lanes = lax.iota(jnp.int32, 16)

    pltpu.sync_copy(idx_hbm.at[pl.ds(wid * IDX_PER_W, IDX_PER_W)], idxw_v)

    def tr_idx(s):
        for b0 in range(0, BB, 16):
            flat = (lanes + b0) * SEQ + s
            idxt_v[s, pl.ds(b0, 16)] = plsc.load_gather(idxw_v, [flat])
    pl.loop(0, SEQ)(tr_idx)

    def start_gather(s, buf):
        return pltpu.async_copy(table_hbm.at[idxt_v.at[s]], wide_v.at[buf],
                                gat_sems[buf])

    start_gather(0, 0)

    def body(s0):
        for b in range(2):
            s = s0 + b
            @pl.when(s + 1 < SEQ)
            def _():
                start_gather(s + 1, 1 - b)
            pltpu.make_async_copy(
                table_hbm.at[idxt_v.at[s]], wide_v.at[b], gat_sems[b]).wait()
            @pl.when(s0 > 0)
            def _():
                pltpu.make_async_copy(
                    tile_v.at[b],
                    out_hbm.at[pl.ds(s, 1), :, pl.ds(wid, 1)],
                    out_sems[b]).wait()
            for d in range(DIM):
                for b0 in range(0, BB, 16):
                    vals = plsc.load_gather(
                        wide_v.at[b], [lanes + b0, jnp.full((16,), d, jnp.int32)])
                    tile_v[b, 0, d // 8, 0, d % 8, pl.ds(b0, 16)] = vals
            pltpu.async_copy(
                tile_v.at[b],
                out_hbm.at[pl.ds(s, 1), :, pl.ds(wid, 1)],
                out_sems[b])

    pl.loop(0, SEQ, step=2)(body)

    for b in range(2):
        s = SEQ - 2 + b
        pltpu.make_async_copy(
            tile_v.at[b],
            out_hbm.at[pl.ds(s, 1), :, pl.ds(wid, 1)],
            out_sems[b]).wait()


def kernel(indices, table):
    idx = indices.reshape(-1).astype(jnp.int32)
    p = _gather_kernel(idx, table)
    return p.transpose(2, 4, 0, 1, 3).reshape(BATCH, SEQ, DIM)


# speedup vs baseline: 1.9781x; 1.9781x over previous
import functools

import jax
import jax.numpy as jnp
from jax import lax
from jax.experimental import pallas as pl
from jax.experimental.pallas import tpu as pltpu
from jax.experimental.pallas import tpu_sc as plsc

VOCAB = 1000000
BATCH = 4096
SEQ = 200
DIM = 64

NUM_CORES = 2
NUM_SUBCORES = 16
NW = NUM_CORES * NUM_SUBCORES
BB = BATCH // NW
IDX_PER_W = BB * SEQ
PITCH = 133

_mesh = plsc.VectorSubcoreMesh(
    core_axis_name="c", subcore_axis_name="s",
    num_cores=NUM_CORES, num_subcores=NUM_SUBCORES,
)


@functools.partial(
    pl.kernel,
    mesh=_mesh,
    out_type=jax.ShapeDtypeStruct((SEQ, 8, NW, 8, BB), jnp.float32),
    scratch_types=[
        pltpu.VMEM((IDX_PER_W,), jnp.int32),
        pltpu.VMEM((SEQ, BB), jnp.int32),
        pltpu.VMEM((2, BB, DIM), jnp.float32),
        pltpu.VMEM((2, 1, 8, 1, 8, PITCH), jnp.float32),
        pltpu.SemaphoreType.DMA,
        pltpu.SemaphoreType.DMA,
        pltpu.SemaphoreType.DMA,
        pltpu.SemaphoreType.DMA,
    ],
    compiler_params=pltpu.CompilerParams(
        use_tc_tiling_on_sc=False, needs_layout_passes=False),
)
def _gather_kernel(idx_hbm, table_hbm, out_hbm, idxw_v, idxt_v, wide_v,
                   tile_v, gat_sem0, gat_sem1, out_sem0, out_sem1):
    wid = lax.axis_index("s") * NUM_CORES + lax.axis_index("c")
    gat_sems = [gat_sem0, gat_sem1]
    out_sems = [out_sem0, out_sem1]
    lanes = lax.iota(jnp.int32, 16)
    zeros16 = jnp.zeros((16,), jnp.int32)
    dvecs = [lanes + d0 for d0 in range(0, DIM, 16)]
    dhi = [d >> 3 for d in dvecs]
    dlo = [d & 7 for d in dvecs]

    pltpu.sync_copy(idx_hbm.at[pl.ds(wid * IDX_PER_W, IDX_PER_W)], idxw_v)

    def tr_idx(s):
        for b0 in range(0, BB, 16):
            flat = (lanes + b0) * SEQ + s
            idxt_v[s, pl.ds(b0, 16)] = plsc.load_gather(idxw_v, [flat])
    pl.loop(0, SEQ)(tr_idx)

    def start_gather(s, buf):
        return pltpu.async_copy(table_hbm.at[idxt_v.at[s]], wide_v.at[buf],
                                gat_sems[buf])

    start_gather(0, 0)

    def out_dst(s):
        return out_hbm.at[pl.ds(s, 1), :, pl.ds(wid, 1)]

    def out_src(buf):
        return tile_v.at[buf, :, :, :, :, pl.ds(0, BB)]

    def body(s0):
        for b in range(2):
            s = s0 + b
            @pl.when(s + 1 < SEQ)
            def _():
                start_gather(s + 1, 1 - b)
            pltpu.make_async_copy(
                table_hbm.at[idxt_v.at[s]], wide_v.at[b], gat_sems[b]).wait()
            @pl.when(s0 > 0)
            def _():
                pltpu.make_async_copy(
                    out_src(b), out_dst(s), out_sems[b]).wait()

            def tr_row(row):
                rvec = zeros16 + row
                for k in range(DIM // 16):
                    vals = wide_v[b, row, pl.ds(k * 16, 16)]
                    plsc.store_scatter(
                        tile_v.at[b],
                        [zeros16, dhi[k], zeros16, dlo[k], rvec], vals)
            pl.loop(0, BB)(tr_row)

            pltpu.async_copy(out_src(b), out_dst(s), out_sems[b])

    pl.loop(0, SEQ, step=2)(body)

    for b in range(2):
        pltpu.make_async_copy(
            out_src(b), out_dst(SEQ - 2 + b), out_sems[b]).wait()


def kernel(indices, table):
    idx = indices.reshape(-1).astype(jnp.int32)
    p = _gather_kernel(idx, table)
    return p.transpose(2, 4, 0, 1, 3).reshape(BATCH, SEQ, DIM)


# speedup vs baseline: 2.0309x; 1.0267x over previous
import functools

import jax
import jax.numpy as jnp
from jax import lax
from jax.experimental import pallas as pl
from jax.experimental.pallas import tpu as pltpu
from jax.experimental.pallas import tpu_sc as plsc

VOCAB = 1000000
BATCH = 4096
SEQ = 200
DIM = 64

NUM_CORES = 2
NUM_SUBCORES = 16
NW = NUM_CORES * NUM_SUBCORES
BB = BATCH // NW
IDX_PER_W = BB * SEQ
PITCH = 133

_mesh = plsc.VectorSubcoreMesh(
    core_axis_name="c", subcore_axis_name="s",
    num_cores=NUM_CORES, num_subcores=NUM_SUBCORES,
)


@functools.partial(
    pl.kernel,
    mesh=_mesh,
    out_type=jax.ShapeDtypeStruct((SEQ, 8 * NW * 8, BB), jnp.float32),
    scratch_types=[
        pltpu.VMEM((IDX_PER_W,), jnp.int32),
        pltpu.VMEM((SEQ, BB), jnp.int32),
        pltpu.VMEM((2, BB, DIM), jnp.float32),
        pltpu.VMEM((2, 1, 8 * 8, PITCH), jnp.float32),
        pltpu.SemaphoreType.DMA,
        pltpu.SemaphoreType.DMA,
        pltpu.SemaphoreType.DMA,
        pltpu.SemaphoreType.DMA,
    ],
    compiler_params=pltpu.CompilerParams(
        use_tc_tiling_on_sc=False, needs_layout_passes=False),
)
def _gather_kernel(idx_hbm, table_hbm, out_hbm, idxw_v, idxt_v, wide_v,
                   tile_v, gat_sem0, gat_sem1, out_sem0, out_sem1):
    wid = lax.axis_index("s") * NUM_CORES + lax.axis_index("c")
    gat_sems = [gat_sem0, gat_sem1]
    out_sems = [out_sem0, out_sem1]
    lanes = lax.iota(jnp.int32, 16)
    zeros16 = jnp.zeros((16,), jnp.int32)
    dvecs = [lanes + d0 for d0 in range(0, DIM, 16)]
    dhi = [d >> 3 for d in dvecs]
    dlo = [d & 7 for d in dvecs]

    pltpu.sync_copy(idx_hbm.at[pl.ds(wid * IDX_PER_W, IDX_PER_W)], idxw_v)

    def tr_idx(s):
        for b0 in range(0, BB, 16):
            flat = (lanes + b0) * SEQ + s
            idxt_v[s, pl.ds(b0, 16)] = plsc.load_gather(idxw_v, [flat])
    pl.loop(0, SEQ)(tr_idx)

    def start_gather(s, buf):
        return pltpu.async_copy(table_hbm.at[idxt_v.at[s]], wide_v.at[buf],
                                gat_sems[buf])

    start_gather(0, 0)

    def out_dst(s, dh):
        return out_hbm.at[pl.ds(s, 1), pl.ds(dh * (NW * 8) + wid * 8, 8), :]

    def out_src(buf, dh):
        return tile_v.at[buf, :, pl.ds(dh * 8, 8), pl.ds(0, BB)]

    def body(s0):
        for b in range(2):
            s = s0 + b
            @pl.when(s + 1 < SEQ)
            def _():
                start_gather(s + 1, 1 - b)
            pltpu.make_async_copy(
                table_hbm.at[idxt_v.at[s]], wide_v.at[b], gat_sems[b]).wait()
            @pl.when(s0 > 0)
            def _():
                for dh in range(8):
                    pltpu.make_async_copy(
                        out_src(b, dh), out_dst(s, dh), out_sems[b]).wait()

            tile2d = tile_v.at[b, 0]
            def tr_row(row):
                rvec = zeros16 + row
                for k in range(DIM // 16):
                    vals = wide_v[b, row, pl.ds(k * 16, 16)]
                    plsc.store_scatter(tile2d, [dvecs[k], rvec], vals)
            pl.loop(0, BB, unroll=8)(tr_row)

            for dh in range(8):
                pltpu.async_copy(out_src(b, dh), out_dst(s, dh), out_sems[b])

    pl.loop(0, SEQ, step=2)(body)

    for b in range(2):
        for dh in range(8):
            pltpu.make_async_copy(
                out_src(b, dh), out_dst(SEQ - 2 + b, dh), out_sems[b]).wait()


def kernel(indices, table):
    idx = indices.reshape(-1).astype(jnp.int32)
    p = _gather_kernel(idx, table)
    p = p.reshape(SEQ, 8, NW, 8, BB)
    return p.transpose(2, 4, 0, 1, 3).reshape(BATCH, SEQ, DIM)
